# baseline (speedup 1.0000x reference)
import functools

import jax
import jax.numpy as jnp
from jax import lax
from jax.experimental import pallas as pl
from jax.experimental.pallas import tpu as pltpu
from jax.experimental.pallas import tpu_sc as plsc

D_MODEL = 768
VOCAB = 100000
BATCH = 4
SEQ = 2048
ROWS = BATCH * SEQ

NC, NS = 2, 16
NW = NC * NS
RPW = ROWS // NW
R = 16
SIZES = [8, 8] + [16] * 14 + [8, 8]
OFFS = [sum(SIZES[:i]) for i in range(len(SIZES))]
NCHUNK = len(SIZES)
NBUF = 8
NFLY = 5


def _gather_rows(tab, x):
    mesh = plsc.VectorSubcoreMesh(core_axis_name="c", subcore_axis_name="s",
                                  num_cores=NC, num_subcores=NS)

    @functools.partial(
        pl.kernel,
        out_type=jax.ShapeDtypeStruct((ROWS, D_MODEL), jnp.float32),
        mesh=mesh,
        scratch_types=[
            pltpu.VMEM((RPW,), jnp.int32),
            [pltpu.VMEM((R, D_MODEL), jnp.float32) for _ in range(NBUF)],
            [pltpu.SemaphoreType.DMA for _ in range(NFLY)],
            [pltpu.SemaphoreType.DMA for _ in range(NBUF)],
        ],
    )
    def k(tab_hbm, x_hbm, out_hbm, x_v, bufs, gsems, ssems):
        wid = lax.axis_index("s") * NC + lax.axis_index("c")
        base = wid * RPW

        def start_gather(c):
            return pltpu.async_copy(
                tab_hbm.at[x_v.at[pl.ds(OFFS[c], SIZES[c])]],
                bufs[c % NBUF].at[pl.ds(0, SIZES[c])], gsems[c % NFLY])

        def start_scatter(c):
            return pltpu.async_copy(
                bufs[c % NBUF].at[pl.ds(0, SIZES[c])],
                out_hbm.at[pl.ds(base + OFFS[c], SIZES[c])],
                ssems[c % NBUF])

        pltpu.sync_copy(x_hbm.at[base // SEQ, pl.ds(base % SEQ, RPW)], x_v)
        gathers = {c: start_gather(c) for c in range(NFLY)}

        scatters = {}
        for c in range(NCHUNK):
            gathers[c].wait()
            scatters[c] = start_scatter(c)
            nxt = c + NFLY
            if nxt < NCHUNK:
                prev = nxt - NBUF
                if prev >= 0:
                    scatters[prev].wait()
                gathers[nxt] = start_gather(nxt)
        for c in range(max(0, NCHUNK - NBUF + NFLY), NCHUNK):
            scatters[c].wait()

    return k(tab, x)


def kernel(x, W_E):
    tab = W_E.T
    out = _gather_rows(tab, x.astype(jnp.int32))
    return out.reshape(x.shape[0], x.shape[1], D_MODEL)


# speedup vs baseline: 1.0054x; 1.0054x over previous
import functools

import jax
import jax.numpy as jnp
from jax import lax
from jax.experimental import pallas as pl
from jax.experimental.pallas import tpu as pltpu
from jax.experimental.pallas import tpu_sc as plsc

D_MODEL = 768
VOCAB = 100000
BATCH = 4
SEQ = 2048
ROWS = BATCH * SEQ

NC, NS = 2, 16
NW = NC * NS
RPW = ROWS // NW
R = 16
SIZES = [8, 8] + [16] * 14 + [8, 8]
OFFS = [sum(SIZES[:i]) for i in range(len(SIZES))]
NCHUNK = len(SIZES)
NBUF = 10
NFLY = 4


def _gather_rows(tab, x):
    mesh = plsc.VectorSubcoreMesh(core_axis_name="c", subcore_axis_name="s",
                                  num_cores=NC, num_subcores=NS)

    @functools.partial(
        pl.kernel,
        out_type=jax.ShapeDtypeStruct((ROWS, D_MODEL), jnp.float32),
        mesh=mesh,
        scratch_types=[
            pltpu.VMEM((RPW,), jnp.int32),
            [pltpu.VMEM((R, D_MODEL), jnp.float32) for _ in range(NBUF)],
            [pltpu.SemaphoreType.DMA for _ in range(NFLY)],
            [pltpu.SemaphoreType.DMA for _ in range(NBUF)],
        ],
    )
    def k(tab_hbm, x_hbm, out_hbm, x_v, bufs, gsems, ssems):
        wid = lax.axis_index("s") * NC + lax.axis_index("c")
        base = wid * RPW

        def start_gather(c):
            return pltpu.async_copy(
                tab_hbm.at[x_v.at[pl.ds(OFFS[c], SIZES[c])]],
                bufs[c % NBUF].at[pl.ds(0, SIZES[c])], gsems[c % NFLY])

        def start_scatter(c):
            return pltpu.async_copy(
                bufs[c % NBUF].at[pl.ds(0, SIZES[c])],
                out_hbm.at[pl.ds(base + OFFS[c], SIZES[c])],
                ssems[c % NBUF])

        pltpu.sync_copy(x_hbm.at[base // SEQ, pl.ds(base % SEQ, RPW)], x_v)
        gathers = {c: start_gather(c) for c in range(NFLY)}

        scatters = {}
        for c in range(NCHUNK):
            gathers[c].wait()
            scatters[c] = start_scatter(c)
            nxt = c + NFLY
            if nxt < NCHUNK:
                prev = nxt - NBUF
                if prev >= 0:
                    scatters[prev].wait()
                gathers[nxt] = start_gather(nxt)
        for c in range(max(0, NCHUNK - NBUF + NFLY), NCHUNK):
            scatters[c].wait()

    return k(tab, x)


def kernel(x, W_E):
    tab = W_E.T
    out = _gather_rows(tab, x.astype(jnp.int32))
    return out.reshape(x.shape[0], x.shape[1], D_MODEL)
